# baseline (speedup 1.0000x reference)
import functools

import jax
import jax.numpy as jnp
from jax import lax
from jax.experimental import pallas as pl
from jax.experimental.pallas import tpu as pltpu
from jax.experimental.pallas import tpu_sc as plsc

NC = 2
NS = 16
L = 16
NW = NC * NS

B = 16384
D = 128
G = 64
PER_W = B // NW
NCHUNK = PER_W // G


def _body(idx_hbm, uemb_hbm, memb_hbm, ubias_hbm, mbias_hbm,
          out_hbm,
          iflat_v, uidx_v, midx_v, urows_v, mrows_v, ubias_v, mbias_v,
          p_v, out_v, sem0, sem1):
  wid = lax.axis_index("s") * NC + lax.axis_index("c")
  base = wid * PER_W

  iot = lax.iota(jnp.int32, L)
  sems = (sem0, sem1)

  pltpu.sync_copy(idx_hbm.at[pl.ds(base * 2, 2 * PER_W)], iflat_v)
  for j in range(NCHUNK):
    for g in range(G // L):
      src = (j * G + g * L) * 2 + iot * 2
      uidx_v[j, pl.ds(g * L, L)] = plsc.load_gather(iflat_v, [src])
      midx_v[j, pl.ds(g * L, L)] = plsc.load_gather(iflat_v, [src + 1])

  def launch(j, b):
    sem = sems[b]
    pltpu.async_copy(uemb_hbm.at[uidx_v.at[j]], urows_v.at[b], sem)
    pltpu.async_copy(memb_hbm.at[midx_v.at[j]], mrows_v.at[b], sem)
    pltpu.async_copy(ubias_hbm.at[uidx_v.at[j]], ubias_v.at[b], sem)
    pltpu.async_copy(mbias_hbm.at[midx_v.at[j]], mbias_v.at[b], sem)

  def drain(j, b):
    sem = sems[b]
    pltpu.make_async_copy(uemb_hbm.at[uidx_v.at[j]], urows_v.at[b], sem).wait()
    pltpu.make_async_copy(memb_hbm.at[midx_v.at[j]], mrows_v.at[b], sem).wait()
    pltpu.make_async_copy(ubias_hbm.at[uidx_v.at[j]], ubias_v.at[b], sem).wait()
    pltpu.make_async_copy(mbias_hbm.at[midx_v.at[j]], mbias_v.at[b], sem).wait()

  def compute(j, b):
    for g in range(G // L):
      for i in range(L):
        row = g * L + i
        acc = urows_v[b, row, pl.ds(0, L)] * mrows_v[b, row, pl.ds(0, L)]
        for k in range(1, D // L):
          acc = acc + urows_v[b, row, pl.ds(k * L, L)] * mrows_v[b, row, pl.ds(k * L, L)]
        p_v[pl.ds(i * L, L)] = acc
      cols = iot * L
      tot = plsc.load_gather(p_v, [cols])
      for c in range(1, L):
        tot = tot + plsc.load_gather(p_v, [cols + c])
      x = tot + ubias_v[b, pl.ds(g * L, L)] + mbias_v[b, pl.ds(g * L, L)]
      y = 5.0 / (1.0 + jnp.exp(-x))
      out_v[pl.ds(j * G + g * L, L)] = y

  launch(0, 0)

  def pair_body(t, carry):
    j0 = 2 * t
    j1 = j0 + 1
    launch(j1, 1)
    drain(j0, 0)
    compute(j0, 0)

    @pl.when(j1 + 1 < NCHUNK)
    def _():
      launch(j1 + 1, 0)

    drain(j1, 1)
    compute(j1, 1)
    return carry

  lax.fori_loop(0, NCHUNK // 2, pair_body, 0, unroll=False)
  pltpu.sync_copy(out_v, out_hbm.at[pl.ds(base, PER_W)])


@functools.partial(jax.jit, donate_argnums=())
def _run(idx_flat, uemb, memb, ubias, mbias):
  mesh = plsc.VectorSubcoreMesh(core_axis_name="c", subcore_axis_name="s",
                                num_cores=NC, num_subcores=NS)
  fn = pl.kernel(
      _body,
      out_type=jax.ShapeDtypeStruct((B,), jnp.float32),
      mesh=mesh,
      compiler_params=pltpu.CompilerParams(needs_layout_passes=False),
      scratch_types=[
          pltpu.VMEM((2 * PER_W,), jnp.int32),
          pltpu.VMEM((NCHUNK, G), jnp.int32),
          pltpu.VMEM((NCHUNK, G), jnp.int32),
          pltpu.VMEM((2, G, D), jnp.float32),
          pltpu.VMEM((2, G, D), jnp.float32),
          pltpu.VMEM((2, G), jnp.float32),
          pltpu.VMEM((2, G), jnp.float32),
          pltpu.VMEM((L * L,), jnp.float32),
          pltpu.VMEM((PER_W,), jnp.float32),
          pltpu.SemaphoreType.DMA,
          pltpu.SemaphoreType.DMA,
      ],
  )
  return fn(idx_flat, uemb, memb, ubias, mbias)


def kernel(inputs, user_emb, user_bias, movie_emb, movie_bias):
  idx_flat = inputs.astype(jnp.int32).reshape(-1)
  out = _run(idx_flat, user_emb, movie_emb,
             user_bias.reshape(-1), movie_bias.reshape(-1))
  return out.reshape(B, 1)


# speedup vs baseline: 1.1257x; 1.1257x over previous
import functools

import jax
import jax.numpy as jnp
from jax import lax
from jax.experimental import pallas as pl
from jax.experimental.pallas import tpu as pltpu
from jax.experimental.pallas import tpu_sc as plsc

NC = 2
NS = 16
L = 16
NW = NC * NS

B = 16384
D = 128
V = 100000
G = 64
PER_W = B // NW
NCHUNK = PER_W // G

UB_OFF = 2 * B
MB_OFF = 2 * B + V


def _body(comb_hbm, uemb_hbm, memb_hbm,
          out_hbm,
          uidx_v, midx_v, ubidx_v, mbidx_v,
          urows_v, mrows_v, ubias_v, mbias_v,
          p_v, out_v, sem0, sem1):
  wid = lax.axis_index("s") * NC + lax.axis_index("c")
  base = wid * PER_W

  iot = lax.iota(jnp.int32, L)
  sems = (sem0, sem1)

  pltpu.sync_copy(comb_hbm.at[pl.ds(base, PER_W)], uidx_v)
  pltpu.sync_copy(comb_hbm.at[pl.ds(B + base, PER_W)], midx_v)
  for g in range(PER_W // L):
    sl = pl.ds(g * L, L)
    ubidx_v[sl] = uidx_v[sl] + UB_OFF
    mbidx_v[sl] = midx_v[sl] + MB_OFF

  def launch(j, b):
    sem = sems[b]
    sl = pl.ds(j * G, G)
    pltpu.async_copy(uemb_hbm.at[uidx_v.at[sl]], urows_v.at[b], sem)
    pltpu.async_copy(memb_hbm.at[midx_v.at[sl]], mrows_v.at[b], sem)
    pltpu.async_copy(comb_hbm.at[ubidx_v.at[sl]], ubias_v.at[b], sem)
    pltpu.async_copy(comb_hbm.at[mbidx_v.at[sl]], mbias_v.at[b], sem)

  def drain(j, b):
    sem = sems[b]
    sl = pl.ds(j * G, G)
    pltpu.make_async_copy(uemb_hbm.at[uidx_v.at[sl]], urows_v.at[b], sem).wait()
    pltpu.make_async_copy(memb_hbm.at[midx_v.at[sl]], mrows_v.at[b], sem).wait()
    pltpu.make_async_copy(comb_hbm.at[ubidx_v.at[sl]], ubias_v.at[b], sem).wait()
    pltpu.make_async_copy(comb_hbm.at[mbidx_v.at[sl]], mbias_v.at[b], sem).wait()

  def compute(j, b):
    for g in range(G // L):
      for i in range(L):
        row = g * L + i
        acc = urows_v[b, row, pl.ds(0, L)] * mrows_v[b, row, pl.ds(0, L)]
        for k in range(1, D // L):
          acc = acc + urows_v[b, row, pl.ds(k * L, L)] * mrows_v[b, row, pl.ds(k * L, L)]
        p_v[pl.ds(i * L, L)] = acc
      cols = iot * L
      tot = plsc.load_gather(p_v, [cols])
      for c in range(1, L):
        tot = tot + plsc.load_gather(p_v, [cols + c])
      ub = plsc.bitcast(ubias_v[b, pl.ds(g * L, L)], jnp.float32)
      mb = plsc.bitcast(mbias_v[b, pl.ds(g * L, L)], jnp.float32)
      x = tot + ub + mb
      y = 5.0 / (1.0 + jnp.exp(-x))
      out_v[pl.ds(j * G + g * L, L)] = y

  launch(0, 0)

  def pair_body(t, carry):
    j0 = 2 * t
    j1 = j0 + 1
    launch(j1, 1)
    drain(j0, 0)
    compute(j0, 0)

    @pl.when(j1 + 1 < NCHUNK)
    def _():
      launch(j1 + 1, 0)

    drain(j1, 1)
    compute(j1, 1)
    return carry

  lax.fori_loop(0, NCHUNK // 2, pair_body, 0, unroll=False)
  pltpu.sync_copy(out_v, out_hbm.at[pl.ds(base, PER_W)])


@functools.partial(jax.jit, donate_argnums=())
def _run(comb, uemb, memb):
  mesh = plsc.VectorSubcoreMesh(core_axis_name="c", subcore_axis_name="s",
                                num_cores=NC, num_subcores=NS)
  fn = pl.kernel(
      _body,
      out_type=jax.ShapeDtypeStruct((B,), jnp.float32),
      mesh=mesh,
      compiler_params=pltpu.CompilerParams(needs_layout_passes=False),
      scratch_types=[
          pltpu.VMEM((PER_W,), jnp.int32),
          pltpu.VMEM((PER_W,), jnp.int32),
          pltpu.VMEM((PER_W,), jnp.int32),
          pltpu.VMEM((PER_W,), jnp.int32),
          pltpu.VMEM((2, G, D), jnp.float32),
          pltpu.VMEM((2, G, D), jnp.float32),
          pltpu.VMEM((2, G), jnp.int32),
          pltpu.VMEM((2, G), jnp.int32),
          pltpu.VMEM((L * L,), jnp.float32),
          pltpu.VMEM((PER_W,), jnp.float32),
          pltpu.SemaphoreType.DMA,
          pltpu.SemaphoreType.DMA,
      ],
  )
  return fn(comb, uemb, memb)


def kernel(inputs, user_emb, user_bias, movie_emb, movie_bias):
  idx = inputs.astype(jnp.int32)
  comb = jnp.concatenate([
      idx[:, 0],
      idx[:, 1],
      lax.bitcast_convert_type(user_bias[:, 0], jnp.int32),
      lax.bitcast_convert_type(movie_bias[:, 0], jnp.int32),
  ])
  out = _run(comb, user_emb, movie_emb)
  return out.reshape(B, 1)
